# baseline (speedup 1.0000x reference)
import jax
import jax.numpy as jnp
from jax import lax
from jax.experimental import pallas as pl
from jax.experimental.pallas import tpu as pltpu, tpu_sc as plsc

VOCAB = 1000000
EMBED_DIM = 32
BATCH = 16384
FIELDS = 26

_B = BATCH * FIELDS
_NW = 32
_PER_W = _B // _NW
_CH = 512
_NCH = _PER_W // _CH
_EHI = EMBED_DIM // 8
_BHI = BATCH // 128
_OW = _EHI * 4 * 8 * 128

_mesh = plsc.VectorSubcoreMesh(core_axis_name="c", subcore_axis_name="s")


@jax.jit
def _run(table, idx):
    @pl.kernel(
        out_type=jax.ShapeDtypeStruct((_B * EMBED_DIM,), jnp.float32),
        mesh=_mesh,
        scratch_types=[
            pltpu.VMEM((_PER_W,), jnp.int32),
            pltpu.VMEM((2 * _CH, EMBED_DIM), jnp.float32),
            pltpu.VMEM((2 * _OW,), jnp.float32),
            pltpu.SemaphoreType.DMA,
            pltpu.SemaphoreType.DMA,
        ],
        compiler_params=pltpu.CompilerParams(
            use_tc_tiling_on_sc=False, needs_layout_passes=False
        ),
    )
    def k(table_hbm, idx_hbm, out_hbm, idx_v, rows2, o2, gsem, wsem):
        wid = lax.axis_index("s") * 2 + lax.axis_index("c")
        w_base = wid * _PER_W
        g_base = wid * (_PER_W // 128)
        pltpu.sync_copy(idx_hbm.at[pl.ds(w_base, _PER_W)], idx_v)

        iota = lax.iota(jnp.int32, 16)

        def start_gather(i, p):
            pltpu.async_copy(
                table_hbm.at[idx_v.at[pl.ds(i * _CH, _CH)]],
                rows2.at[pl.ds(p * _CH, _CH)],
                gsem,
            )

        def wait_gather():
            pltpu.make_async_copy(
                table_hbm.at[idx_v.at[pl.ds(0, _CH)]],
                rows2.at[pl.ds(0, _CH)],
                gsem,
            ).wait()

        def wait_write():
            pltpu.make_async_copy(
                out_hbm.at[pl.ds(0, 4096)],
                o2.at[pl.ds(0, 4096)],
                wsem,
            ).wait()

        start_gather(0, 0)

        def body(i, _):
            p = i & 1
            wait_gather()

            @pl.when(i + 1 < _NCH)
            def _():
                start_gather(i + 1, 1 - p)

            @pl.when(i >= 2)
            def _():
                for _w in range(_EHI):
                    wait_write()

            rows = rows2.at[pl.ds(p * _CH, _CH), :]
            o = o2.at[pl.ds(p * _OW, _OW)]
            for ehi in range(_EHI):
                for blk in range(4):
                    for elo in range(8):
                        col = jnp.full((16,), ehi * 8 + elo, jnp.int32)
                        obase = (ehi * 32 + blk * 8 + elo) * 128
                        for j in range(8):
                            row = iota + (blk * 128 + 16 * j)
                            vec = plsc.load_gather(rows, [row, col])
                            o[pl.ds(obase + 16 * j, 16)] = vec

            g = g_base + 4 * i
            f = g >> 7
            bhi0 = g & 127
            for ehi in range(_EHI):
                pltpu.async_copy(
                    o.at[pl.ds(ehi * 4096, 4096)],
                    out_hbm.at[pl.ds(
                        ((f * _EHI + ehi) * _BHI + bhi0) * 1024, 4096)],
                    wsem,
                )
            return 0

        lax.fori_loop(0, _NCH, body, 0)
        for _i in range(2 * _EHI):
            wait_write()

    return k(table, idx)


def kernel(batch, emb_weight):
    idx = batch.T.reshape(_B).astype(jnp.int32)
    o5 = _run(emb_weight, idx).reshape(FIELDS, _EHI, _BHI, 8, 128)
    return o5.transpose(2, 4, 0, 1, 3).reshape(BATCH, FIELDS, EMBED_DIM)


# speedup vs baseline: 1.2302x; 1.2302x over previous
import jax
import jax.numpy as jnp
from jax import lax
from jax.experimental import pallas as pl
from jax.experimental.pallas import tpu as pltpu, tpu_sc as plsc

VOCAB = 1000000
EMBED_DIM = 32
BATCH = 16384
FIELDS = 26

_B = BATCH * FIELDS
_NW = 32
_PER_W = _B // _NW
_CH = 512
_NCH = _PER_W // _CH
_EHI = EMBED_DIM // 8
_BHI = BATCH // 128
_OW = _EHI * 4 * 8 * 128

_mesh = plsc.VectorSubcoreMesh(core_axis_name="c", subcore_axis_name="s")


@jax.jit
def _run(table, idx):
    @pl.kernel(
        out_type=jax.ShapeDtypeStruct((_B * EMBED_DIM,), jnp.float32),
        mesh=_mesh,
        scratch_types=[
            pltpu.VMEM((_PER_W,), jnp.int32),
            pltpu.VMEM((2 * _CH, EMBED_DIM), jnp.float32),
            pltpu.VMEM((2 * _OW,), jnp.float32),
            pltpu.SemaphoreType.DMA,
            pltpu.SemaphoreType.DMA,
        ],
        compiler_params=pltpu.CompilerParams(
            use_tc_tiling_on_sc=False, needs_layout_passes=False
        ),
    )
    def k(table_hbm, idx_hbm, out_hbm, idx_v, rows2, o2, gsem, wsem):
        wid = lax.axis_index("s") * 2 + lax.axis_index("c")
        w_base = wid * _PER_W
        g_base = wid * (_PER_W // 128)
        pltpu.sync_copy(idx_hbm.at[pl.ds(w_base, _PER_W)], idx_v)

        iota = lax.iota(jnp.int32, 16)

        def start_gather(i, p):
            pltpu.async_copy(
                table_hbm.at[idx_v.at[pl.ds(i * _CH, _CH)]],
                rows2.at[pl.ds(p * _CH, _CH)],
                gsem,
            )

        def wait_gather():
            pltpu.make_async_copy(
                table_hbm.at[idx_v.at[pl.ds(0, _CH)]],
                rows2.at[pl.ds(0, _CH)],
                gsem,
            ).wait()

        def wait_write():
            pltpu.make_async_copy(
                out_hbm.at[pl.ds(0, 4096)],
                o2.at[pl.ds(0, 4096)],
                wsem,
            ).wait()

        start_gather(0, 0)

        def body(i, _):
            p = i & 1
            wait_gather()

            @pl.when(i + 1 < _NCH)
            def _():
                start_gather(i + 1, 1 - p)

            @pl.when(i >= 2)
            def _():
                for _w in range(_EHI):
                    wait_write()

            rows = rows2.at[pl.ds(p * _CH, _CH), :]
            o = o2.at[pl.ds(p * _OW, _OW)]

            @plsc.parallel_loop(0, 128, unroll=4)
            def _t(m):
                ehi = m >> 5
                blk = (m >> 3) & 3
                elo = m & 7
                col = jnp.full((16,), ehi * 8 + elo, jnp.int32)
                rowb = blk * 128
                obase = m * 128
                for j in range(8):
                    row = iota + (rowb + 16 * j)
                    vec = plsc.load_gather(rows, [row, col])
                    o[pl.ds(obase + 16 * j, 16)] = vec

            g = g_base + 4 * i
            f = g >> 7
            bhi0 = g & 127
            for ehi in range(_EHI):
                pltpu.async_copy(
                    o.at[pl.ds(ehi * 4096, 4096)],
                    out_hbm.at[pl.ds(
                        ((f * _EHI + ehi) * _BHI + bhi0) * 1024, 4096)],
                    wsem,
                )
            return 0

        lax.fori_loop(0, _NCH, body, 0)
        for _i in range(2 * _EHI):
            wait_write()

    return k(table, idx)


def kernel(batch, emb_weight):
    idx = batch.T.reshape(_B).astype(jnp.int32)
    o5 = _run(emb_weight, idx).reshape(FIELDS, _EHI, _BHI, 8, 128)
    return o5.transpose(2, 4, 0, 1, 3).reshape(BATCH, FIELDS, EMBED_DIM)


# speedup vs baseline: 1.2494x; 1.0156x over previous
import jax
import jax.numpy as jnp
from jax import lax
from jax.experimental import pallas as pl
from jax.experimental.pallas import tpu as pltpu, tpu_sc as plsc

VOCAB = 1000000
EMBED_DIM = 32
BATCH = 16384
FIELDS = 26

_B = BATCH * FIELDS
_NW = 32
_PER_W = _B // _NW
_CH = 256
_NCH = _PER_W // _CH
_EHI = EMBED_DIM // 8
_BHI = BATCH // 128
_OW = _EHI * 2 * 8 * 128

_mesh = plsc.VectorSubcoreMesh(core_axis_name="c", subcore_axis_name="s")


@jax.jit
def _run(table, idx):
    @pl.kernel(
        out_type=jax.ShapeDtypeStruct((_B * EMBED_DIM,), jnp.float32),
        mesh=_mesh,
        scratch_types=[
            pltpu.VMEM((_PER_W,), jnp.int32),
            pltpu.VMEM((2 * _CH, 128), jnp.float32),
            pltpu.VMEM((2 * _OW,), jnp.float32),
            pltpu.SemaphoreType.DMA,
            pltpu.SemaphoreType.DMA,
        ],
        compiler_params=pltpu.CompilerParams(
            use_tc_tiling_on_sc=False, needs_layout_passes=False
        ),
    )
    def k(table_hbm, idx_hbm, out_hbm, idx_v, rows2, o2, gsem, wsem):
        wid = lax.axis_index("s") * 2 + lax.axis_index("c")
        w_base = wid * _PER_W
        g_base = wid * (_PER_W // 128)
        pltpu.sync_copy(idx_hbm.at[pl.ds(w_base, _PER_W)], idx_v)

        iota = lax.iota(jnp.int32, 16)

        def start_gather(i, p):
            pltpu.async_copy(
                table_hbm.at[idx_v.at[pl.ds(i * _CH, _CH)]],
                rows2.at[pl.ds(p * _CH, _CH)],
                gsem,
            )

        def wait_gather():
            pltpu.make_async_copy(
                table_hbm.at[idx_v.at[pl.ds(0, _CH)]],
                rows2.at[pl.ds(0, _CH)],
                gsem,
            ).wait()

        def wait_write():
            pltpu.make_async_copy(
                out_hbm.at[pl.ds(0, 2048)],
                o2.at[pl.ds(0, 2048)],
                wsem,
            ).wait()

        start_gather(0, 0)

        def body(i, _):
            p = i & 1
            wait_gather()

            @pl.when(i + 1 < _NCH)
            def _():
                start_gather(i + 1, 1 - p)

            @pl.when(i >= 2)
            def _():
                for _w in range(_EHI):
                    wait_write()

            rows = rows2.at[pl.ds(p * _CH, _CH), :]
            o = o2.at[pl.ds(p * _OW, _OW)]

            @plsc.parallel_loop(0, 64, unroll=4)
            def _t(m):
                ehi = m >> 4
                blk = (m >> 3) & 1
                elo = m & 7
                col = jnp.full((16,), ehi * 8 + elo, jnp.int32)
                rowb = blk * 128
                obase = m * 128
                for j in range(8):
                    row = iota + (rowb + 16 * j)
                    vec = plsc.load_gather(rows, [row, col])
                    o[pl.ds(obase + 16 * j, 16)] = vec

            g = g_base + 2 * i
            f = g >> 7
            bhi0 = g & 127
            for ehi in range(_EHI):
                pltpu.async_copy(
                    o.at[pl.ds(ehi * 2048, 2048)],
                    out_hbm.at[pl.ds(
                        ((f * _EHI + ehi) * _BHI + bhi0) * 1024, 2048)],
                    wsem,
                )
            return 0

        lax.fori_loop(0, _NCH, body, 0)
        for _i in range(2 * _EHI):
            wait_write()

    return k(table, idx)


def kernel(batch, emb_weight):
    idx = batch.T.reshape(_B).astype(jnp.int32)
    padded = jnp.pad(emb_weight, ((0, 0), (0, 128 - EMBED_DIM)))
    o5 = _run(padded, idx).reshape(FIELDS, _EHI, _BHI, 8, 128)
    return o5.transpose(2, 4, 0, 1, 3).reshape(BATCH, FIELDS, EMBED_DIM)
